# initial kernel scaffold (unmeasured)
import jax
import jax.numpy as jnp
from jax import lax
from jax.experimental import pallas as pl
from jax.experimental.pallas import tpu as pltpu


def kernel(
    x,
):
    def body(*refs):
        pass

    out_shape = jax.ShapeDtypeStruct(..., jnp.float32)
    return pl.pallas_call(body, out_shape=out_shape)(...)


# baseline (device time: 60055 ns/iter reference)
import jax
import jax.numpy as jnp
from jax import lax
from jax.experimental import pallas as pl
from jax.experimental.pallas import tpu as pltpu

N_DEV = 4
BM = 1024


def kernel(x):
    m, n = x.shape
    grid = m // BM

    def body(x_hbm, cur, nxt, out_ref, halo_top, halo_bot, prev_last,
             send_sems, recv_sems):
        i = pl.program_id(0)
        my_pos = lax.axis_index("i")
        left = my_pos - 1
        right = my_pos + 1
        last_step = grid - 1

        @pl.when((i == 0) & (my_pos < N_DEV - 1))
        def _():
            rdma = pltpu.make_async_remote_copy(
                src_ref=x_hbm.at[pl.ds(m - 1, 1)],
                dst_ref=halo_top,
                send_sem=send_sems.at[0],
                recv_sem=recv_sems.at[0],
                device_id=(right,),
                device_id_type=pl.DeviceIdType.MESH,
            )
            rdma.start()

        @pl.when((i == 0) & (my_pos > 0))
        def _():
            rdma = pltpu.make_async_remote_copy(
                src_ref=x_hbm.at[pl.ds(0, 1)],
                dst_ref=halo_bot,
                send_sem=send_sems.at[1],
                recv_sem=recv_sems.at[1],
                device_id=(left,),
                device_id_type=pl.DeviceIdType.MESH,
            )
            rdma.start()

        out_ref[1 : BM - 1, :] = (
            0.25 * cur[0 : BM - 2, :]
            + 0.5 * cur[1 : BM - 1, :]
            + 0.25 * cur[2:BM, :]
        )

        @pl.when(i < last_step)
        def _():
            out_ref[BM - 1 : BM, :] = (
                0.25 * cur[BM - 2 : BM - 1, :]
                + 0.5 * cur[BM - 1 : BM, :]
                + 0.25 * nxt[0:1, :]
            )

        @pl.when((i == last_step) & (my_pos < N_DEV - 1))
        def _():
            recv = pltpu.make_async_remote_copy(
                src_ref=x_hbm.at[pl.ds(0, 1)],
                dst_ref=halo_bot,
                send_sem=send_sems.at[1],
                recv_sem=recv_sems.at[1],
                device_id=(right,),
                device_id_type=pl.DeviceIdType.MESH,
            )
            recv.wait_recv()
            out_ref[BM - 1 : BM, :] = (
                0.25 * cur[BM - 2 : BM - 1, :]
                + 0.5 * cur[BM - 1 : BM, :]
                + 0.25 * halo_bot[:, :]
            )

        @pl.when((i == last_step) & (my_pos == N_DEV - 1))
        def _():
            out_ref[BM - 1 : BM, :] = cur[BM - 1 : BM, :]

        @pl.when(i > 0)
        def _():
            out_ref[0:1, :] = (
                0.25 * prev_last[:, :] + 0.5 * cur[0:1, :] + 0.25 * cur[1:2, :]
            )

        @pl.when((i == 0) & (my_pos > 0))
        def _():
            recv = pltpu.make_async_remote_copy(
                src_ref=x_hbm.at[pl.ds(0, 1)],
                dst_ref=halo_top,
                send_sem=send_sems.at[0],
                recv_sem=recv_sems.at[0],
                device_id=(left,),
                device_id_type=pl.DeviceIdType.MESH,
            )
            recv.wait_recv()
            out_ref[0:1, :] = (
                0.25 * halo_top[:, :] + 0.5 * cur[0:1, :] + 0.25 * cur[1:2, :]
            )

        @pl.when((i == 0) & (my_pos == 0))
        def _():
            out_ref[0:1, :] = cur[0:1, :]

        prev_last[:, :] = cur[BM - 1 : BM, :]

        @pl.when((i == last_step) & (my_pos < N_DEV - 1))
        def _():
            send = pltpu.make_async_remote_copy(
                src_ref=x_hbm.at[pl.ds(m - 1, 1)],
                dst_ref=halo_top,
                send_sem=send_sems.at[0],
                recv_sem=recv_sems.at[0],
                device_id=(right,),
                device_id_type=pl.DeviceIdType.MESH,
            )
            send.wait_send()

        @pl.when((i == last_step) & (my_pos > 0))
        def _():
            send = pltpu.make_async_remote_copy(
                src_ref=x_hbm.at[pl.ds(0, 1)],
                dst_ref=halo_bot,
                send_sem=send_sems.at[1],
                recv_sem=recv_sems.at[1],
                device_id=(left,),
                device_id_type=pl.DeviceIdType.MESH,
            )
            send.wait_send()

    return pl.pallas_call(
        body,
        grid=(grid,),
        out_shape=jax.ShapeDtypeStruct((m, n), x.dtype),
        in_specs=[
            pl.BlockSpec(memory_space=pltpu.MemorySpace.HBM),
            pl.BlockSpec((BM, n), lambda i: (i, 0)),
            pl.BlockSpec((BM, n), lambda i: (jnp.minimum(i + 1, grid - 1), 0)),
        ],
        out_specs=pl.BlockSpec((BM, n), lambda i: (i, 0)),
        scratch_shapes=[
            pltpu.VMEM((1, n), x.dtype),
            pltpu.VMEM((1, n), x.dtype),
            pltpu.VMEM((1, n), x.dtype),
            pltpu.SemaphoreType.DMA((2,)),
            pltpu.SemaphoreType.DMA((2,)),
        ],
    )(x, x, x)
